# baseline (speedup 1.0000x reference)
import functools

import jax
import jax.numpy as jnp
from jax import lax
from jax.experimental import pallas as pl
from jax.experimental.pallas import tpu as pltpu
from jax.experimental.pallas import tpu_sc as plsc

VOCAB = 100000
D = 1024
BATCH = 4
SEQ = 8192

NC = 2
NS = 16
NW = NC * NS

POS_PER_W = SEQ // NW
CHUNK = 16
NCHUNK = POS_PER_W // CHUNK
NITEM = BATCH * NCHUNK
RING = 5
LOOKAHEAD = 3
LANES = 16
SLICES = D // LANES


def _make_sc_kernel():
    mesh = plsc.VectorSubcoreMesh(core_axis_name="c", subcore_axis_name="s")

    @functools.partial(
        pl.kernel,
        mesh=mesh,
        out_type=jax.ShapeDtypeStruct((BATCH * SEQ, D), jnp.float32),
        scratch_types=[
            pltpu.VMEM((BATCH * NCHUNK, CHUNK), jnp.int32),
            pltpu.VMEM((RING, CHUNK, D), jnp.float32),
            pltpu.VMEM((2, CHUNK, D), jnp.float32),
            pltpu.SemaphoreType.DMA((RING,)),
            pltpu.SemaphoreType.DMA((RING,)),
            pltpu.SemaphoreType.DMA((2,)),
        ],
    )
    def emb_kernel(x2d_hbm, table_hbm, pe_hbm, out_hbm,
                   idx_v, rows_v, pe_v, gsem, ssem, psem):
        wid = lax.axis_index("s") * NC + lax.axis_index("c")
        pos0 = wid * POS_PER_W

        for b in range(BATCH):
            pltpu.sync_copy(
                x2d_hbm.at[pl.ds(b * (SEQ // CHUNK) + wid * NCHUNK, NCHUNK)],
                idx_v.at[pl.ds(b * NCHUNK, NCHUNK)],
            )

        def g_copy(t):
            c, b, slot = t // BATCH, lax.rem(t, BATCH), lax.rem(t, RING)
            return pltpu.make_async_copy(
                table_hbm.at[idx_v.at[b * NCHUNK + c]], rows_v.at[slot],
                gsem.at[slot])

        def s_copy(t):
            c, b, slot = t // BATCH, lax.rem(t, BATCH), lax.rem(t, RING)
            return pltpu.make_async_copy(
                rows_v.at[slot],
                out_hbm.at[pl.ds(b * SEQ + pos0 + c * CHUNK, CHUNK)],
                ssem.at[slot])

        def pe_copy(c):
            return pltpu.make_async_copy(
                pe_hbm.at[pl.ds(pos0 + c * CHUNK, CHUNK)],
                pe_v.at[lax.rem(c, 2)], psem.at[lax.rem(c, 2)])

        pe_copy(0).start()
        for tp in range(LOOKAHEAD):
            g_copy(tp).start()

        def item(t, carry):
            c = t // BATCH
            b = lax.rem(t, BATCH)

            @pl.when(b == 0)
            def _():
                pe_copy(c).wait()

                @pl.when(c < NCHUNK - 1)
                def _():
                    pe_copy(c + 1).start()

            @pl.when(t >= RING - LOOKAHEAD)
            def _():
                s_copy(t - (RING - LOOKAHEAD)).wait()

            @pl.when(t < NITEM - LOOKAHEAD)
            def _():
                g_copy(t + LOOKAHEAD).start()

            g_copy(t).wait()

            slot = lax.rem(t, RING)
            rows_ref = rows_v.at[slot]
            pe_ref = pe_v.at[lax.rem(c, 2)]

            @plsc.parallel_loop(0, CHUNK, 1, unroll=4)
            def _(r):
                for j in range(SLICES):
                    sl = pl.ds(j * LANES, LANES)
                    plsc.addupdate(rows_ref.at[r, sl], pe_ref[r, sl])

            s_copy(t).start()
            return carry

        lax.fori_loop(0, NITEM, item, 0, unroll=False)

        for dt in range(NITEM - (RING - LOOKAHEAD), NITEM):
            s_copy(dt).wait()

    return emb_kernel


_emb_kernel = _make_sc_kernel()


def kernel(x, table, pe):
    x2d = x.reshape(BATCH * SEQ // CHUNK, CHUNK)
    out = _emb_kernel(x2d, table, pe)
    return out.reshape(BATCH, SEQ, D)


# speedup vs baseline: 1.1483x; 1.1483x over previous
import functools

import jax
import jax.numpy as jnp
from jax import lax
from jax.experimental import pallas as pl
from jax.experimental.pallas import tpu as pltpu
from jax.experimental.pallas import tpu_sc as plsc

VOCAB = 100000
D = 1024
BATCH = 4
SEQ = 8192

NC = 2
NS = 16
NW = NC * NS

POS_PER_W = SEQ // NW
CHUNK = 16
NCHUNK = POS_PER_W // CHUNK
NITEM = BATCH * NCHUNK
RING = 5
LOOKAHEAD = 3
LANES = 16
SLICES = D // LANES


def _make_sc_kernel():
    mesh = plsc.VectorSubcoreMesh(core_axis_name="c", subcore_axis_name="s")

    @functools.partial(
        pl.kernel,
        mesh=mesh,
        out_type=jax.ShapeDtypeStruct((BATCH * SEQ, D), jnp.float32),
        scratch_types=[
            pltpu.VMEM((BATCH * NCHUNK, CHUNK), jnp.int32),
            pltpu.VMEM((RING, CHUNK, D), jnp.float32),
            pltpu.VMEM((2, CHUNK, D), jnp.float32),
            pltpu.SemaphoreType.DMA((RING,)),
            pltpu.SemaphoreType.DMA((RING,)),
            pltpu.SemaphoreType.DMA((2,)),
        ],
    )
    def emb_kernel(x2d_hbm, table_hbm, pe_hbm, out_hbm,
                   idx_v, rows_v, pe_v, gsem, ssem, psem):
        wid = lax.axis_index("s") * NC + lax.axis_index("c")
        pos0 = wid * POS_PER_W

        for b in range(BATCH):
            pltpu.sync_copy(
                x2d_hbm.at[pl.ds(b * (SEQ // CHUNK) + wid * NCHUNK, NCHUNK)],
                idx_v.at[pl.ds(b * NCHUNK, NCHUNK)],
            )

        def g_copy(t):
            c, b, slot = t // BATCH, lax.rem(t, BATCH), lax.rem(t, RING)
            return pltpu.make_async_copy(
                table_hbm.at[idx_v.at[b * NCHUNK + c]], rows_v.at[slot],
                gsem.at[slot])

        def s_copy(t):
            c, b, slot = t // BATCH, lax.rem(t, BATCH), lax.rem(t, RING)
            return pltpu.make_async_copy(
                rows_v.at[slot],
                out_hbm.at[pl.ds(b * SEQ + pos0 + c * CHUNK, CHUNK)],
                ssem.at[slot])

        def pe_copy(c):
            return pltpu.make_async_copy(
                pe_hbm.at[pl.ds(pos0 + c * CHUNK, CHUNK)],
                pe_v.at[lax.rem(c, 2)], psem.at[lax.rem(c, 2)])

        pe_copy(0).start()
        for tp in range(LOOKAHEAD):
            g_copy(tp).start()

        def item(t, carry):
            c = t // BATCH
            b = lax.rem(t, BATCH)

            @pl.when(b == 0)
            def _():
                pe_copy(c).wait()

                @pl.when(c < NCHUNK - 1)
                def _():
                    pe_copy(c + 1).start()

            @pl.when(t >= RING - LOOKAHEAD)
            def _():
                s_copy(t - (RING - LOOKAHEAD)).wait()

            @pl.when(t < NITEM - LOOKAHEAD)
            def _():
                g_copy(t + LOOKAHEAD).start()

            g_copy(t).wait()

            slot = lax.rem(t, RING)
            rows_ref = rows_v.at[slot]
            pe_ref = pe_v.at[lax.rem(c, 2)]

            @plsc.parallel_loop(0, CHUNK, 1, unroll=2)
            def _(r):
                for j in range(SLICES):
                    sl = pl.ds(j * LANES, LANES)
                    plsc.addupdate(rows_ref.at[r, sl], pe_ref[r, sl])

            s_copy(t).start()
            return carry

        lax.fori_loop(0, NITEM, item, 0, unroll=False)

        for dt in range(NITEM - (RING - LOOKAHEAD), NITEM):
            s_copy(dt).wait()

    return emb_kernel


_emb_kernel = _make_sc_kernel()


def kernel(x, table, pe):
    x2d = x.reshape(BATCH * SEQ // CHUNK, CHUNK)
    out = _emb_kernel(x2d, table, pe)
    return out.reshape(BATCH, SEQ, D)
